# baseline (speedup 1.0000x reference)
import functools

import jax
import jax.numpy as jnp
from jax import lax
from jax.experimental import pallas as pl
from jax.experimental.pallas import tpu as pltpu
from jax.experimental.pallas import tpu_sc as plsc

N = 10000
NP = 10240
D = 128
E = 320000
EP = 327680
NW = 32
CHUNK = 128
NCHUNK = EP // (NW * CHUNK)
ROWS_PER_TILE = NP // 16

_MESH = dict(core_axis_name="c", subcore_axis_name="s", num_cores=2,
             num_subcores=16)


def _deg_body(dst_hbm, deg_out, idx_v, buf_v, deg_sh, sem_v):
    c = lax.axis_index("c")
    s = lax.axis_index("s")
    wid = c * 16 + s

    def _z(i, _):
        buf_v[pl.ds(i * 16, 16)] = jnp.zeros((16,), jnp.float32)
        return _

    lax.fori_loop(0, ROWS_PER_TILE // 16, _z, None)
    pltpu.sync_copy(buf_v, deg_sh.at[pl.ds(s * ROWS_PER_TILE, ROWS_PER_TILE)])
    plsc.subcore_barrier()

    def _o(i, _):
        buf_v[pl.ds(i * 16, 16)] = jnp.ones((16,), jnp.float32)
        return _

    lax.fori_loop(0, CHUNK // 16, _o, None)

    pltpu.sync_copy(dst_hbm.at[wid], idx_v)
    ones = buf_v.at[pl.ds(0, CHUNK)]

    def _grp(g, _):
        for i in range(8):
            pltpu.async_copy(ones, deg_sh.at[idx_v.at[g * 8 + i]], sem_v,
                             add=True)
        for i in range(8):
            pltpu.make_async_copy(ones, deg_sh.at[idx_v.at[g * 8 + i]],
                                  sem_v).wait()
        return _

    lax.fori_loop(0, NCHUNK // 8, _grp, None)
    plsc.subcore_barrier()

    pltpu.sync_copy(deg_sh.at[pl.ds(s * ROWS_PER_TILE, ROWS_PER_TILE)],
                    deg_out.at[c, pl.ds(s * ROWS_PER_TILE, ROWS_PER_TILE)])


_deg_call = functools.partial(
    pl.kernel,
    out_type=jax.ShapeDtypeStruct((2, NP), jnp.float32),
    mesh=plsc.VectorSubcoreMesh(**_MESH),
    scratch_types=[
        pltpu.VMEM((NCHUNK, CHUNK), jnp.int32),
        pltpu.VMEM((ROWS_PER_TILE,), jnp.float32),
        pltpu.VMEM_SHARED((NP,), jnp.float32),
        pltpu.SemaphoreType.DMA,
    ],
)(_deg_body)


def _mm_body(x_ref, w_ref, deg_ref, hp_ref):
    d = deg_ref[0, :] + deg_ref[1, :]
    dinv = lax.rsqrt(jnp.maximum(d, 1.0))
    h = jnp.dot(x_ref[...], w_ref[...], preferred_element_type=jnp.float32)
    hp_ref[...] = h * dinv[:, None]


def _mm_call(x_pad, w, deg2):
    return pl.pallas_call(
        _mm_body,
        grid=(NP // 512,),
        in_specs=[
            pl.BlockSpec((512, D), lambda i: (i, 0)),
            pl.BlockSpec((D, D), lambda i: (0, 0)),
            pl.BlockSpec((2, 512), lambda i: (0, i)),
        ],
        out_specs=pl.BlockSpec((512, D), lambda i: (i, 0)),
        out_shape=jax.ShapeDtypeStruct((NP, D), jnp.float32),
    )(x_pad, w, deg2)


HALF = NCHUNK // 2


def _scat_body(hp_hbm, src_hbm, dst_hbm, acc_out, sidx, didx, buf0, buf1,
               acc_sh, sem0, sem1):
    c = lax.axis_index("c")
    s = lax.axis_index("s")
    wid = c * 16 + s
    bufs = (buf0, buf1)
    sems = (sem0, sem1)

    for k in range(ROWS_PER_TILE // CHUNK):
        pltpu.async_copy(hp_hbm.at[pl.ds(NP - CHUNK, CHUNK)],
                         acc_sh.at[pl.ds(s * ROWS_PER_TILE + k * CHUNK, CHUNK)],
                         sem0)
    for k in range(ROWS_PER_TILE // CHUNK):
        pltpu.make_async_copy(
            hp_hbm.at[pl.ds(NP - CHUNK, CHUNK)],
            acc_sh.at[pl.ds(s * ROWS_PER_TILE + k * CHUNK, CHUNK)],
            sem0).wait()
    plsc.subcore_barrier()

    for h in range(2):
        pltpu.sync_copy(src_hbm.at[wid, pl.ds(h * HALF, HALF)], sidx)
        pltpu.sync_copy(dst_hbm.at[wid, pl.ds(h * HALF, HALF)], didx)

        pltpu.async_copy(hp_hbm.at[sidx.at[0]], buf0, sem0)
        pltpu.async_copy(hp_hbm.at[sidx.at[1]], buf1, sem1)

        def _pair(k, _):
            for b in range(2):
                j = 2 * k + b
                pltpu.make_async_copy(hp_hbm.at[sidx.at[j]], bufs[b],
                                      sems[b]).wait()
                pltpu.sync_copy(bufs[b], acc_sh.at[didx.at[j]], add=True)

                @pl.when(j + 2 < HALF)
                def _():
                    pltpu.async_copy(hp_hbm.at[sidx.at[j + 2]], bufs[b],
                                     sems[b])
            return _

        lax.fori_loop(0, HALF // 2, _pair, None)
    plsc.subcore_barrier()

    r0 = s * ROWS_PER_TILE
    pltpu.sync_copy(acc_sh.at[pl.ds(r0, ROWS_PER_TILE)],
                    acc_out.at[c, pl.ds(r0, ROWS_PER_TILE)])


_scat_call = functools.partial(
    pl.kernel,
    out_type=jax.ShapeDtypeStruct((2, NP, D), jnp.float32),
    mesh=plsc.VectorSubcoreMesh(**_MESH),
    scratch_types=[
        pltpu.VMEM((HALF, CHUNK), jnp.int32),
        pltpu.VMEM((HALF, CHUNK), jnp.int32),
        pltpu.VMEM((CHUNK, D), jnp.float32),
        pltpu.VMEM((CHUNK, D), jnp.float32),
        pltpu.VMEM_SHARED((NP, D), jnp.float32),
        pltpu.SemaphoreType.DMA,
        pltpu.SemaphoreType.DMA,
    ],
)(_scat_body)


def _fin_body(acc_ref, deg_ref, b_ref, out_ref):
    d = deg_ref[0, :] + deg_ref[1, :]
    dinv = lax.rsqrt(jnp.maximum(d, 1.0))
    acc = acc_ref[0] + acc_ref[1]
    out_ref[...] = jnp.maximum(acc * dinv[:, None] + b_ref[...], 0.0)


def _fin_call(acc2, deg2, b):
    return pl.pallas_call(
        _fin_body,
        grid=(NP // 512,),
        in_specs=[
            pl.BlockSpec((2, 512, D), lambda i: (0, i, 0)),
            pl.BlockSpec((2, 512), lambda i: (0, i)),
            pl.BlockSpec((D,), lambda i: (0,)),
        ],
        out_specs=pl.BlockSpec((512, D), lambda i: (i, 0)),
        out_shape=jax.ShapeDtypeStruct((NP, D), jnp.float32),
    )(acc2, deg2, b)


def kernel(x, edge, edge_type, edge_norm, W, b):
    del edge_type, edge_norm
    src = edge[0]
    dst = edge[1]
    pad = EP - E
    pad_ids = (jnp.arange(pad, dtype=jnp.int32) % (NP - N)) + N
    src3 = jnp.concatenate([src, pad_ids]).reshape(NW, NCHUNK, CHUNK)
    dst3 = jnp.concatenate([dst, pad_ids]).reshape(NW, NCHUNK, CHUNK)
    x_pad = jnp.pad(x, ((0, NP - N), (0, 0)))

    deg2 = _deg_call(dst3)
    hp = _mm_call(x_pad, W, deg2)
    acc2 = _scat_call(hp, src3, dst3)
    outp = _fin_call(acc2, deg2, b)
    return outp[:N]


# speedup vs baseline: 1.0528x; 1.0528x over previous
import functools

import jax
import jax.numpy as jnp
from jax import lax
from jax.experimental import pallas as pl
from jax.experimental.pallas import tpu as pltpu
from jax.experimental.pallas import tpu_sc as plsc

N = 10000
NP = 10240
D = 128
E = 320000
EP = 327680
NW = 32
CHUNK = 128
NCHUNK = EP // (NW * CHUNK)
ROWS_PER_TILE = NP // 16

_MESH = dict(core_axis_name="c", subcore_axis_name="s", num_cores=2,
             num_subcores=16)


def _deg_body(dst_hbm, deg_out, idx_v, buf_v, deg_sh, sem_v):
    c = lax.axis_index("c")
    s = lax.axis_index("s")
    wid = c * 16 + s

    def _z(i, _):
        buf_v[pl.ds(i * 16, 16)] = jnp.zeros((16,), jnp.float32)
        return _

    lax.fori_loop(0, ROWS_PER_TILE // 16, _z, None)
    pltpu.sync_copy(buf_v, deg_sh.at[pl.ds(s * ROWS_PER_TILE, ROWS_PER_TILE)])
    plsc.subcore_barrier()

    def _o(i, _):
        buf_v[pl.ds(i * 16, 16)] = jnp.ones((16,), jnp.float32)
        return _

    lax.fori_loop(0, CHUNK // 16, _o, None)

    pltpu.sync_copy(dst_hbm.at[wid], idx_v)
    ones = buf_v.at[pl.ds(0, CHUNK)]

    def _grp(g, _):
        for i in range(8):
            pltpu.async_copy(ones, deg_sh.at[idx_v.at[g * 8 + i]], sem_v,
                             add=True)
        for i in range(8):
            pltpu.make_async_copy(ones, deg_sh.at[idx_v.at[g * 8 + i]],
                                  sem_v).wait()
        return _

    lax.fori_loop(0, NCHUNK // 8, _grp, None)
    plsc.subcore_barrier()

    pltpu.sync_copy(deg_sh.at[pl.ds(s * ROWS_PER_TILE, ROWS_PER_TILE)],
                    deg_out.at[c, pl.ds(s * ROWS_PER_TILE, ROWS_PER_TILE)])


_deg_call = functools.partial(
    pl.kernel,
    out_type=jax.ShapeDtypeStruct((2, NP), jnp.float32),
    mesh=plsc.VectorSubcoreMesh(**_MESH),
    scratch_types=[
        pltpu.VMEM((NCHUNK, CHUNK), jnp.int32),
        pltpu.VMEM((ROWS_PER_TILE,), jnp.float32),
        pltpu.VMEM_SHARED((NP,), jnp.float32),
        pltpu.SemaphoreType.DMA,
    ],
)(_deg_body)


def _mm_body(x_ref, w_ref, deg_ref, hp_ref):
    d = deg_ref[0, :] + deg_ref[1, :]
    dinv = lax.rsqrt(jnp.maximum(d, 1.0))
    h = jnp.dot(x_ref[...], w_ref[...], preferred_element_type=jnp.float32)
    hp_ref[...] = h * dinv[:, None]


def _mm_call(x_pad, w, deg2):
    return pl.pallas_call(
        _mm_body,
        grid=(NP // 512,),
        in_specs=[
            pl.BlockSpec((512, D), lambda i: (i, 0)),
            pl.BlockSpec((D, D), lambda i: (0, 0)),
            pl.BlockSpec((2, 512), lambda i: (0, i)),
        ],
        out_specs=pl.BlockSpec((512, D), lambda i: (i, 0)),
        out_shape=jax.ShapeDtypeStruct((NP, D), jnp.float32),
    )(x_pad, w, deg2)


HALF = NCHUNK // 2


def _scat_body(hp_hbm, src_hbm, dst_hbm, acc_out, sidx, didx, buf0, buf1,
               acc_sh, sem0, sem1):
    c = lax.axis_index("c")
    s = lax.axis_index("s")
    wid = c * 16 + s
    bufs = (buf0, buf1)
    sems = (sem0, sem1)

    def _z(r, _):
        for k in range(D // 16):
            buf0[r, pl.ds(k * 16, 16)] = jnp.zeros((16,), jnp.float32)
        return _

    lax.fori_loop(0, CHUNK, _z, None)
    for k in range(ROWS_PER_TILE // CHUNK):
        pltpu.sync_copy(buf0,
                        acc_sh.at[pl.ds(s * ROWS_PER_TILE + k * CHUNK, CHUNK)])
    plsc.subcore_barrier()

    for h in range(2):
        pltpu.sync_copy(src_hbm.at[wid, pl.ds(h * HALF, HALF)], sidx)
        pltpu.sync_copy(dst_hbm.at[wid, pl.ds(h * HALF, HALF)], didx)

        pltpu.async_copy(hp_hbm.at[sidx.at[0]], buf0, sem0)
        pltpu.async_copy(hp_hbm.at[sidx.at[1]], buf1, sem1)

        def _pair(k, _):
            for b in range(2):
                j = 2 * k + b
                pltpu.make_async_copy(hp_hbm.at[sidx.at[j]], bufs[b],
                                      sems[b]).wait()
                pltpu.sync_copy(bufs[b], acc_sh.at[didx.at[j]], add=True)

                @pl.when(j + 2 < HALF)
                def _():
                    pltpu.async_copy(hp_hbm.at[sidx.at[j + 2]], bufs[b],
                                     sems[b])
            return _

        lax.fori_loop(0, HALF // 2, _pair, None)
    plsc.subcore_barrier()

    r0 = s * ROWS_PER_TILE
    pltpu.sync_copy(acc_sh.at[pl.ds(r0, ROWS_PER_TILE)],
                    acc_out.at[c, pl.ds(r0, ROWS_PER_TILE)])


_scat_call = functools.partial(
    pl.kernel,
    out_type=jax.ShapeDtypeStruct((2, NP, D), jnp.float32),
    mesh=plsc.VectorSubcoreMesh(**_MESH),
    scratch_types=[
        pltpu.VMEM((HALF, CHUNK), jnp.int32),
        pltpu.VMEM((HALF, CHUNK), jnp.int32),
        pltpu.VMEM((CHUNK, D), jnp.float32),
        pltpu.VMEM((CHUNK, D), jnp.float32),
        pltpu.VMEM_SHARED((NP, D), jnp.float32),
        pltpu.SemaphoreType.DMA,
        pltpu.SemaphoreType.DMA,
    ],
)(_scat_body)


def _fin_body(acc_ref, deg_ref, b_ref, out_ref):
    d = deg_ref[0, :] + deg_ref[1, :]
    dinv = lax.rsqrt(jnp.maximum(d, 1.0))
    acc = acc_ref[0] + acc_ref[1]
    out_ref[...] = jnp.maximum(acc * dinv[:, None] + b_ref[...], 0.0)


def _fin_call(acc2, deg2, b):
    return pl.pallas_call(
        _fin_body,
        grid=(NP // 512,),
        in_specs=[
            pl.BlockSpec((2, 512, D), lambda i: (0, i, 0)),
            pl.BlockSpec((2, 512), lambda i: (0, i)),
            pl.BlockSpec((D,), lambda i: (0,)),
        ],
        out_specs=pl.BlockSpec((512, D), lambda i: (i, 0)),
        out_shape=jax.ShapeDtypeStruct((NP, D), jnp.float32),
    )(acc2, deg2, b)


def kernel(x, edge, edge_type, edge_norm, W, b):
    del edge_type, edge_norm
    src = edge[0]
    dst = edge[1]
    pad = EP - E
    pad_ids = (jnp.arange(pad, dtype=jnp.int32) % (NP - N)) + N
    src3 = jnp.concatenate([src, pad_ids]).reshape(NW, NCHUNK, CHUNK)
    dst3 = jnp.concatenate([dst, pad_ids]).reshape(NW, NCHUNK, CHUNK)
    x_pad = jnp.pad(x, ((0, NP - N), (0, 0)))

    deg2 = _deg_call(dst3)
    hp = _mm_call(x_pad, W, deg2)
    acc2 = _scat_call(hp, src3, dst3)
    outp = _fin_call(acc2, deg2, b)
    return outp[:N]


# speedup vs baseline: 1.1215x; 1.0652x over previous
import functools

import jax
import jax.numpy as jnp
from jax import lax
from jax.experimental import pallas as pl
from jax.experimental.pallas import tpu as pltpu
from jax.experimental.pallas import tpu_sc as plsc

N = 10000
NP = 10240
D = 128
E = 320000
CHUNK = 128
TOT_CHUNKS = E // CHUNK
BASE = TOT_CHUNKS // 32
EXTRA_W = TOT_CHUNKS - 32 * BASE
PH_A = 40
PH_B = BASE - PH_A
ROWS_PER_TILE = NP // 16

_MESH = dict(core_axis_name="c", subcore_axis_name="s", num_cores=2,
             num_subcores=16)


def _build_rowlist(rl, base, n):
    for i in range((n + 15) // 16):
        rl[pl.ds(i * 16, 16)] = base + i * 16 + lax.iota(jnp.int32, 16)


def _deg_body(dst_hbm, deg_out, idx_v, buf_v, rl_v, deg_sh, sem_v):
    c = lax.axis_index("c")
    s = lax.axis_index("s")
    wid = c * 16 + s

    def _z(i, _):
        buf_v[pl.ds(i * 16, 16)] = jnp.zeros((16,), jnp.float32)
        return _

    lax.fori_loop(0, ROWS_PER_TILE // 16, _z, None)
    pltpu.sync_copy(buf_v, deg_sh.at[pl.ds(s * ROWS_PER_TILE, ROWS_PER_TILE)])
    plsc.subcore_barrier()

    def _o(i, _):
        buf_v[pl.ds(i * 16, 16)] = jnp.ones((16,), jnp.float32)
        return _

    lax.fori_loop(0, CHUNK // 16, _o, None)
    ones = buf_v.at[pl.ds(0, CHUNK)]

    for ph, (off, n) in enumerate(((0, PH_A), (PH_A, PH_B))):
        _build_rowlist(rl_v, wid * BASE + off, n)
        pltpu.sync_copy(dst_hbm.at[rl_v.at[pl.ds(0, n)]],
                        idx_v.at[pl.ds(0, n)])

        def _grp(g, _):
            for i in range(2):
                pltpu.async_copy(ones, deg_sh.at[idx_v.at[g * 2 + i]], sem_v,
                                 add=True)
            for i in range(2):
                pltpu.make_async_copy(ones, deg_sh.at[idx_v.at[g * 2 + i]],
                                      sem_v).wait()
            return _

        lax.fori_loop(0, n // 2, _grp, None)

    @pl.when(wid < EXTRA_W // 2)
    def _():
        _build_rowlist(rl_v, 32 * BASE + 2 * wid, 2)
        pltpu.sync_copy(dst_hbm.at[rl_v.at[pl.ds(0, 2)]],
                        idx_v.at[pl.ds(0, 2)])
        for j in range(2):
            pltpu.sync_copy(ones, deg_sh.at[idx_v.at[j]], add=True)

    plsc.subcore_barrier()

    pltpu.sync_copy(deg_sh.at[pl.ds(s * ROWS_PER_TILE, ROWS_PER_TILE)],
                    deg_out.at[c, pl.ds(s * ROWS_PER_TILE, ROWS_PER_TILE)])


_deg_call = functools.partial(
    pl.kernel,
    out_type=jax.ShapeDtypeStruct((2, NP), jnp.float32),
    mesh=plsc.VectorSubcoreMesh(**_MESH),
    scratch_types=[
        pltpu.VMEM((PH_A, CHUNK), jnp.int32),
        pltpu.VMEM((ROWS_PER_TILE,), jnp.float32),
        pltpu.VMEM((48,), jnp.int32),
        pltpu.VMEM_SHARED((NP,), jnp.float32),
        pltpu.SemaphoreType.DMA,
    ],
)(_deg_body)


def _mm_body(x_ref, w_ref, deg_ref, hp_ref):
    d = deg_ref[0, :] + deg_ref[1, :]
    dinv = lax.rsqrt(jnp.maximum(d, 1.0))
    h = jnp.dot(x_ref[...], w_ref[...], preferred_element_type=jnp.float32)
    hp_ref[...] = h * dinv[:, None]


def _mm_call(x, w, deg2):
    return pl.pallas_call(
        _mm_body,
        grid=(NP // 1024,),
        in_specs=[
            pl.BlockSpec((1024, D), lambda i: (i, 0)),
            pl.BlockSpec((D, D), lambda i: (0, 0)),
            pl.BlockSpec((2, 1024), lambda i: (0, i)),
        ],
        out_specs=pl.BlockSpec((1024, D), lambda i: (i, 0)),
        out_shape=jax.ShapeDtypeStruct((NP, D), jnp.float32),
    )(x, w, deg2)


def _scat_body(hp_hbm, src_hbm, dst_hbm, acc_out, sidx, didx, buf0, buf1,
               rl_v, acc_sh, sem0, sem1):
    c = lax.axis_index("c")
    s = lax.axis_index("s")
    wid = c * 16 + s
    bufs = (buf0, buf1)
    sems = (sem0, sem1)

    def _z(r, _):
        for k in range(D // 16):
            buf0[r, pl.ds(k * 16, 16)] = jnp.zeros((16,), jnp.float32)
        return _

    lax.fori_loop(0, CHUNK, _z, None)
    for k in range(ROWS_PER_TILE // CHUNK):
        pltpu.sync_copy(buf0,
                        acc_sh.at[pl.ds(s * ROWS_PER_TILE + k * CHUNK, CHUNK)])
    plsc.subcore_barrier()

    for ph, (off, n) in enumerate(((0, PH_A), (PH_A, PH_B))):
        _build_rowlist(rl_v, wid * BASE + off, n)
        pltpu.async_copy(src_hbm.at[rl_v.at[pl.ds(0, n)]],
                         sidx.at[pl.ds(0, n)], sem0)
        pltpu.async_copy(dst_hbm.at[rl_v.at[pl.ds(0, n)]],
                         didx.at[pl.ds(0, n)], sem0)
        pltpu.make_async_copy(src_hbm.at[rl_v.at[pl.ds(0, n)]],
                              sidx.at[pl.ds(0, n)], sem0).wait()
        pltpu.make_async_copy(dst_hbm.at[rl_v.at[pl.ds(0, n)]],
                              didx.at[pl.ds(0, n)], sem0).wait()

        pltpu.async_copy(hp_hbm.at[sidx.at[0]], buf0, sem0)
        pltpu.async_copy(hp_hbm.at[sidx.at[1]], buf1, sem1)

        def _pair(k, _):
            for b in range(2):
                j = 2 * k + b
                pltpu.make_async_copy(hp_hbm.at[sidx.at[j]], bufs[b],
                                      sems[b]).wait()
                pltpu.sync_copy(bufs[b], acc_sh.at[didx.at[j]], add=True)

                @pl.when(j + 2 < n)
                def _():
                    pltpu.async_copy(hp_hbm.at[sidx.at[j + 2]], bufs[b],
                                     sems[b])
            return _

        lax.fori_loop(0, n // 2, _pair, None)

    @pl.when(wid < EXTRA_W // 2)
    def _():
        _build_rowlist(rl_v, 32 * BASE + 2 * wid, 2)
        pltpu.sync_copy(src_hbm.at[rl_v.at[pl.ds(0, 2)]],
                        sidx.at[pl.ds(0, 2)])
        pltpu.sync_copy(dst_hbm.at[rl_v.at[pl.ds(0, 2)]],
                        didx.at[pl.ds(0, 2)])
        for j in range(2):
            pltpu.sync_copy(hp_hbm.at[sidx.at[j]], buf0)
            pltpu.sync_copy(buf0, acc_sh.at[didx.at[j]], add=True)

    plsc.subcore_barrier()

    r0 = s * ROWS_PER_TILE
    pltpu.sync_copy(acc_sh.at[pl.ds(r0, ROWS_PER_TILE)],
                    acc_out.at[c, pl.ds(r0, ROWS_PER_TILE)])


_scat_call = functools.partial(
    pl.kernel,
    out_type=jax.ShapeDtypeStruct((2, NP, D), jnp.float32),
    mesh=plsc.VectorSubcoreMesh(**_MESH),
    scratch_types=[
        pltpu.VMEM((PH_A, CHUNK), jnp.int32),
        pltpu.VMEM((PH_A, CHUNK), jnp.int32),
        pltpu.VMEM((CHUNK, D), jnp.float32),
        pltpu.VMEM((CHUNK, D), jnp.float32),
        pltpu.VMEM((48,), jnp.int32),
        pltpu.VMEM_SHARED((NP, D), jnp.float32),
        pltpu.SemaphoreType.DMA,
        pltpu.SemaphoreType.DMA,
    ],
)(_scat_body)


def _fin_body(acc_ref, deg_ref, b_ref, out_ref):
    d = deg_ref[0, :] + deg_ref[1, :]
    dinv = lax.rsqrt(jnp.maximum(d, 1.0))
    acc = acc_ref[0] + acc_ref[1]
    out_ref[...] = jnp.maximum(acc * dinv[:, None] + b_ref[...], 0.0)


def _fin_call(acc2, deg2, b):
    return pl.pallas_call(
        _fin_body,
        grid=(NP // 1024,),
        in_specs=[
            pl.BlockSpec((2, 1024, D), lambda i: (0, i, 0)),
            pl.BlockSpec((2, 1024), lambda i: (0, i)),
            pl.BlockSpec((D,), lambda i: (0,)),
        ],
        out_specs=pl.BlockSpec((1024, D), lambda i: (i, 0)),
        out_shape=jax.ShapeDtypeStruct((N, D), jnp.float32),
    )(acc2, deg2, b)


def kernel(x, edge, edge_type, edge_norm, W, b):
    del edge_type, edge_norm
    src2 = edge[0].reshape(TOT_CHUNKS, CHUNK)
    dst2 = edge[1].reshape(TOT_CHUNKS, CHUNK)

    deg2 = _deg_call(dst2)
    hp = _mm_call(x, W, deg2)
    acc2 = _scat_call(hp, src2, dst2)
    return _fin_call(acc2, deg2, b)


# speedup vs baseline: 1.1824x; 1.0544x over previous
import functools

import jax
import jax.numpy as jnp
from jax import lax
from jax.experimental import pallas as pl
from jax.experimental.pallas import tpu as pltpu
from jax.experimental.pallas import tpu_sc as plsc

N = 10000
NP = 10240
D = 128
E = 320000
CHUNK = 128
TOT_CHUNKS = E // CHUNK
BASE = TOT_CHUNKS // 32
EXTRA = TOT_CHUNKS - 32 * BASE
PH_A = 40
PH_B = BASE - PH_A
ROWS_PER_TILE = NP // 16

_MESH = dict(core_axis_name="c", subcore_axis_name="s", num_cores=2,
             num_subcores=16)


def _build_didx(didx2, est, n):
    def _cp(i, _):
        j = i // (CHUNK // 16)
        k = i % (CHUNK // 16)
        didx2[j, pl.ds(k * 16, 16)] = est[1, pl.ds(i * 16, 16)]
        return _

    lax.fori_loop(0, n * (CHUNK // 16), _cp, None)


def _deg_body(edge_hbm, deg_out, est, didx2, buf_v, deg_sh, sem_v):
    c = lax.axis_index("c")
    s = lax.axis_index("s")
    wid = c * 16 + s

    def _z(i, _):
        buf_v[pl.ds(i * 16, 16)] = jnp.zeros((16,), jnp.float32)
        return _

    lax.fori_loop(0, ROWS_PER_TILE // 16, _z, None)
    pltpu.sync_copy(buf_v, deg_sh.at[pl.ds(s * ROWS_PER_TILE, ROWS_PER_TILE)])
    plsc.subcore_barrier()

    def _o(i, _):
        buf_v[pl.ds(i * 16, 16)] = jnp.ones((16,), jnp.float32)
        return _

    lax.fori_loop(0, CHUNK // 16, _o, None)
    ones = buf_v.at[pl.ds(0, CHUNK)]

    for ph, (off, n) in enumerate(((0, PH_A), (PH_A, PH_B))):
        pltpu.sync_copy(edge_hbm.at[:, pl.ds((wid * BASE + off) * CHUNK,
                                             n * CHUNK)],
                        est.at[:, pl.ds(0, n * CHUNK)])
        _build_didx(didx2, est, n)

        def _grp(g, _):
            for i in range(2):
                pltpu.async_copy(ones, deg_sh.at[didx2.at[g * 2 + i]], sem_v,
                                 add=True)
            for i in range(2):
                pltpu.make_async_copy(ones, deg_sh.at[didx2.at[g * 2 + i]],
                                      sem_v).wait()
            return _

        lax.fori_loop(0, n // 2, _grp, None)

    @pl.when(wid < EXTRA // 2)
    def _():
        pltpu.sync_copy(edge_hbm.at[:, pl.ds((32 * BASE + 2 * wid) * CHUNK,
                                             2 * CHUNK)],
                        est.at[:, pl.ds(0, 2 * CHUNK)])
        _build_didx(didx2, est, 2)
        for j in range(2):
            pltpu.sync_copy(ones, deg_sh.at[didx2.at[j]], add=True)

    plsc.subcore_barrier()

    pltpu.sync_copy(deg_sh.at[pl.ds(s * ROWS_PER_TILE, ROWS_PER_TILE)],
                    deg_out.at[c, pl.ds(s * ROWS_PER_TILE, ROWS_PER_TILE)])


_deg_call = functools.partial(
    pl.kernel,
    out_type=jax.ShapeDtypeStruct((2, NP), jnp.float32),
    mesh=plsc.VectorSubcoreMesh(**_MESH),
    scratch_types=[
        pltpu.VMEM((2, PH_A * CHUNK), jnp.int32),
        pltpu.VMEM((PH_A, CHUNK), jnp.int32),
        pltpu.VMEM((ROWS_PER_TILE,), jnp.float32),
        pltpu.VMEM_SHARED((NP,), jnp.float32),
        pltpu.SemaphoreType.DMA,
    ],
)(_deg_body)


def _mm_body(x_ref, w_ref, deg_ref, hp_ref):
    d = deg_ref[0, :] + deg_ref[1, :]
    dinv = lax.rsqrt(jnp.maximum(d, 1.0))
    h = jnp.dot(x_ref[...], w_ref[...], preferred_element_type=jnp.float32)
    hp_ref[...] = h * dinv[:, None]


def _mm_call(x, w, deg2):
    return pl.pallas_call(
        _mm_body,
        grid=(NP // 1024,),
        in_specs=[
            pl.BlockSpec((1024, D), lambda i: (i, 0)),
            pl.BlockSpec((D, D), lambda i: (0, 0)),
            pl.BlockSpec((2, 1024), lambda i: (0, i)),
        ],
        out_specs=pl.BlockSpec((1024, D), lambda i: (i, 0)),
        out_shape=jax.ShapeDtypeStruct((NP, D), jnp.float32),
    )(x, w, deg2)


def _scat_body(hp_hbm, edge_hbm, acc_out, est, didx2, buf0, buf1,
               acc_sh, sem0, sem1):
    c = lax.axis_index("c")
    s = lax.axis_index("s")
    wid = c * 16 + s
    bufs = (buf0, buf1)
    sems = (sem0, sem1)

    def _z(r, _):
        for k in range(D // 16):
            buf0[r, pl.ds(k * 16, 16)] = jnp.zeros((16,), jnp.float32)
        return _

    lax.fori_loop(0, CHUNK, _z, None)
    for k in range(ROWS_PER_TILE // CHUNK):
        pltpu.sync_copy(buf0,
                        acc_sh.at[pl.ds(s * ROWS_PER_TILE + k * CHUNK, CHUNK)])
    plsc.subcore_barrier()

    for ph, (off, n) in enumerate(((0, PH_A), (PH_A, PH_B))):
        pltpu.sync_copy(edge_hbm.at[:, pl.ds((wid * BASE + off) * CHUNK,
                                             n * CHUNK)],
                        est.at[:, pl.ds(0, n * CHUNK)])
        _build_didx(didx2, est, n)

        def _sidx(j):
            return est.at[0, pl.ds(j * CHUNK, CHUNK)]

        pltpu.async_copy(hp_hbm.at[_sidx(0)], buf0, sem0)
        pltpu.async_copy(hp_hbm.at[_sidx(1)], buf1, sem1)

        def _pair(k, _):
            for b in range(2):
                j = 2 * k + b
                pltpu.make_async_copy(hp_hbm.at[_sidx(j)], bufs[b],
                                      sems[b]).wait()
                pltpu.sync_copy(bufs[b], acc_sh.at[didx2.at[j]], add=True)

                @pl.when(j + 2 < n)
                def _():
                    pltpu.async_copy(hp_hbm.at[_sidx(j + 2)], bufs[b],
                                     sems[b])
            return _

        lax.fori_loop(0, n // 2, _pair, None)

    @pl.when(wid < EXTRA // 2)
    def _():
        pltpu.sync_copy(edge_hbm.at[:, pl.ds((32 * BASE + 2 * wid) * CHUNK,
                                             2 * CHUNK)],
                        est.at[:, pl.ds(0, 2 * CHUNK)])
        _build_didx(didx2, est, 2)
        for j in range(2):
            pltpu.sync_copy(hp_hbm.at[est.at[0, pl.ds(j * CHUNK, CHUNK)]],
                            buf0)
            pltpu.sync_copy(buf0, acc_sh.at[didx2.at[j]], add=True)

    plsc.subcore_barrier()

    r0 = s * ROWS_PER_TILE
    pltpu.sync_copy(acc_sh.at[pl.ds(r0, ROWS_PER_TILE)],
                    acc_out.at[c, pl.ds(r0, ROWS_PER_TILE)])


_scat_call = functools.partial(
    pl.kernel,
    out_type=jax.ShapeDtypeStruct((2, NP, D), jnp.float32),
    mesh=plsc.VectorSubcoreMesh(**_MESH),
    scratch_types=[
        pltpu.VMEM((2, PH_A * CHUNK), jnp.int32),
        pltpu.VMEM((PH_A, CHUNK), jnp.int32),
        pltpu.VMEM((CHUNK, D), jnp.float32),
        pltpu.VMEM((CHUNK, D), jnp.float32),
        pltpu.VMEM_SHARED((NP, D), jnp.float32),
        pltpu.SemaphoreType.DMA,
        pltpu.SemaphoreType.DMA,
    ],
)(_scat_body)


def _fin_body(acc_ref, deg_ref, b_ref, out_ref):
    d = deg_ref[0, :] + deg_ref[1, :]
    dinv = lax.rsqrt(jnp.maximum(d, 1.0))
    acc = acc_ref[0] + acc_ref[1]
    out_ref[...] = jnp.maximum(acc * dinv[:, None] + b_ref[...], 0.0)


def _fin_call(acc2, deg2, b):
    return pl.pallas_call(
        _fin_body,
        grid=(NP // 1024,),
        in_specs=[
            pl.BlockSpec((2, 1024, D), lambda i: (0, i, 0)),
            pl.BlockSpec((2, 1024), lambda i: (0, i)),
            pl.BlockSpec((D,), lambda i: (0,)),
        ],
        out_specs=pl.BlockSpec((1024, D), lambda i: (i, 0)),
        out_shape=jax.ShapeDtypeStruct((N, D), jnp.float32),
    )(acc2, deg2, b)


def kernel(x, edge, edge_type, edge_norm, W, b):
    del edge_type, edge_norm
    deg2 = _deg_call(edge)
    hp = _mm_call(x, W, deg2)
    acc2 = _scat_call(hp, edge)
    return _fin_call(acc2, deg2, b)


# speedup vs baseline: 1.2252x; 1.0362x over previous
import functools

import jax
import jax.numpy as jnp
from jax import lax
from jax.experimental import pallas as pl
from jax.experimental.pallas import tpu as pltpu
from jax.experimental.pallas import tpu_sc as plsc

N = 10000
NP = 10240
D = 128
E = 320000
CHUNK = 128
TOT_CHUNKS = E // CHUNK
BASE = TOT_CHUNKS // 32
EXTRA = TOT_CHUNKS - 32 * BASE
PH_A = 40
PH_B = BASE - PH_A
ROWS_PER_TILE = NP // 16

_MESH = dict(core_axis_name="c", subcore_axis_name="s", num_cores=2,
             num_subcores=16)


def _build_didx(didx2, est, n):
    def _cp(j, _):
        for k in range(CHUNK // 16):
            didx2[j, pl.ds(k * 16, 16)] = est[1, pl.ds(j * CHUNK + k * 16, 16)]
        return _

    lax.fori_loop(0, n, _cp, None)


def _deg_body(edge_hbm, deg_out, est, didx2, buf_v, deg_sh, sem_v):
    c = lax.axis_index("c")
    s = lax.axis_index("s")
    wid = c * 16 + s

    def _z(i, _):
        buf_v[pl.ds(i * 16, 16)] = jnp.zeros((16,), jnp.float32)
        return _

    lax.fori_loop(0, ROWS_PER_TILE // 16, _z, None)
    pltpu.sync_copy(buf_v, deg_sh.at[pl.ds(s * ROWS_PER_TILE, ROWS_PER_TILE)])
    plsc.subcore_barrier()

    def _o(i, _):
        buf_v[pl.ds(i * 16, 16)] = jnp.ones((16,), jnp.float32)
        return _

    lax.fori_loop(0, CHUNK // 16, _o, None)
    ones = buf_v.at[pl.ds(0, CHUNK)]

    pltpu.sync_copy(edge_hbm.at[:, pl.ds(wid * BASE * CHUNK, BASE * CHUNK)],
                    est.at[:, pl.ds(0, BASE * CHUNK)])
    _build_didx(didx2, est, BASE)

    def _grp(g, _):
        for i in range(2):
            pltpu.async_copy(ones, deg_sh.at[didx2.at[g * 2 + i]], sem_v,
                             add=True)
        for i in range(2):
            pltpu.make_async_copy(ones, deg_sh.at[didx2.at[g * 2 + i]],
                                  sem_v).wait()
        return _

    lax.fori_loop(0, BASE // 2, _grp, None)

    @pl.when(wid < EXTRA // 2)
    def _():
        pltpu.sync_copy(edge_hbm.at[:, pl.ds((32 * BASE + 2 * wid) * CHUNK,
                                             2 * CHUNK)],
                        est.at[:, pl.ds(0, 2 * CHUNK)])
        _build_didx(didx2, est, 2)
        for j in range(2):
            pltpu.sync_copy(ones, deg_sh.at[didx2.at[j]], add=True)

    plsc.subcore_barrier()

    pltpu.sync_copy(deg_sh.at[pl.ds(s * ROWS_PER_TILE, ROWS_PER_TILE)],
                    deg_out.at[c, pl.ds(s * ROWS_PER_TILE, ROWS_PER_TILE)])


_deg_call = functools.partial(
    pl.kernel,
    out_type=jax.ShapeDtypeStruct((2, NP), jnp.float32),
    mesh=plsc.VectorSubcoreMesh(**_MESH),
    scratch_types=[
        pltpu.VMEM((2, BASE * CHUNK), jnp.int32),
        pltpu.VMEM((BASE, CHUNK), jnp.int32),
        pltpu.VMEM((ROWS_PER_TILE,), jnp.float32),
        pltpu.VMEM_SHARED((NP,), jnp.float32),
        pltpu.SemaphoreType.DMA,
    ],
)(_deg_body)


def _mm_body(x_ref, w_ref, deg_ref, hp_ref):
    d = deg_ref[0, :] + deg_ref[1, :]
    dinv = lax.rsqrt(jnp.maximum(d, 1.0))
    h = jnp.dot(x_ref[...], w_ref[...], preferred_element_type=jnp.float32)
    hp_ref[...] = h * dinv[:, None]


def _mm_call(x, w, deg2):
    return pl.pallas_call(
        _mm_body,
        grid=(NP // 2048,),
        in_specs=[
            pl.BlockSpec((2048, D), lambda i: (i, 0)),
            pl.BlockSpec((D, D), lambda i: (0, 0)),
            pl.BlockSpec((2, 2048), lambda i: (0, i)),
        ],
        out_specs=pl.BlockSpec((2048, D), lambda i: (i, 0)),
        out_shape=jax.ShapeDtypeStruct((NP, D), jnp.float32),
    )(x, w, deg2)


def _scat_body(hp_hbm, edge_hbm, acc_out, est, didx2, buf0, buf1,
               acc_sh, sem0, sem1):
    c = lax.axis_index("c")
    s = lax.axis_index("s")
    wid = c * 16 + s
    bufs = (buf0, buf1)
    sems = (sem0, sem1)

    def _z(r, _):
        for k in range(D // 16):
            buf0[r, pl.ds(k * 16, 16)] = jnp.zeros((16,), jnp.float32)
        return _

    lax.fori_loop(0, CHUNK, _z, None)
    for k in range(ROWS_PER_TILE // CHUNK):
        pltpu.sync_copy(buf0,
                        acc_sh.at[pl.ds(s * ROWS_PER_TILE + k * CHUNK, CHUNK)])
    plsc.subcore_barrier()

    for ph, (off, n) in enumerate(((0, PH_A), (PH_A, PH_B))):
        pltpu.sync_copy(edge_hbm.at[:, pl.ds((wid * BASE + off) * CHUNK,
                                             n * CHUNK)],
                        est.at[:, pl.ds(0, n * CHUNK)])
        _build_didx(didx2, est, n)

        def _sidx(j):
            return est.at[0, pl.ds(j * CHUNK, CHUNK)]

        pltpu.async_copy(hp_hbm.at[_sidx(0)], buf0, sem0)
        pltpu.async_copy(hp_hbm.at[_sidx(1)], buf1, sem1)

        def _pair(k, _):
            for b in range(2):
                j = 2 * k + b
                pltpu.make_async_copy(hp_hbm.at[_sidx(j)], bufs[b],
                                      sems[b]).wait()
                pltpu.sync_copy(bufs[b], acc_sh.at[didx2.at[j]], add=True)

                @pl.when(j + 2 < n)
                def _():
                    pltpu.async_copy(hp_hbm.at[_sidx(j + 2)], bufs[b],
                                     sems[b])
            return _

        lax.fori_loop(0, n // 2, _pair, None)

    @pl.when(wid < EXTRA // 2)
    def _():
        pltpu.sync_copy(edge_hbm.at[:, pl.ds((32 * BASE + 2 * wid) * CHUNK,
                                             2 * CHUNK)],
                        est.at[:, pl.ds(0, 2 * CHUNK)])
        _build_didx(didx2, est, 2)
        for j in range(2):
            pltpu.sync_copy(hp_hbm.at[est.at[0, pl.ds(j * CHUNK, CHUNK)]],
                            buf0)
            pltpu.sync_copy(buf0, acc_sh.at[didx2.at[j]], add=True)

    plsc.subcore_barrier()

    r0 = s * ROWS_PER_TILE
    pltpu.sync_copy(acc_sh.at[pl.ds(r0, ROWS_PER_TILE)],
                    acc_out.at[c, pl.ds(r0, ROWS_PER_TILE)])


_scat_call = functools.partial(
    pl.kernel,
    out_type=jax.ShapeDtypeStruct((2, NP, D), jnp.float32),
    mesh=plsc.VectorSubcoreMesh(**_MESH),
    scratch_types=[
        pltpu.VMEM((2, PH_A * CHUNK), jnp.int32),
        pltpu.VMEM((PH_A, CHUNK), jnp.int32),
        pltpu.VMEM((CHUNK, D), jnp.float32),
        pltpu.VMEM((CHUNK, D), jnp.float32),
        pltpu.VMEM_SHARED((NP, D), jnp.float32),
        pltpu.SemaphoreType.DMA,
        pltpu.SemaphoreType.DMA,
    ],
)(_scat_body)


def _fin_body(acc_ref, deg_ref, b_ref, out_ref):
    d = deg_ref[0, :] + deg_ref[1, :]
    dinv = lax.rsqrt(jnp.maximum(d, 1.0))
    acc = acc_ref[0] + acc_ref[1]
    out_ref[...] = jnp.maximum(acc * dinv[:, None] + b_ref[...], 0.0)


def _fin_call(acc2, deg2, b):
    return pl.pallas_call(
        _fin_body,
        grid=(NP // 2048,),
        in_specs=[
            pl.BlockSpec((2, 2048, D), lambda i: (0, i, 0)),
            pl.BlockSpec((2, 2048), lambda i: (0, i)),
            pl.BlockSpec((D,), lambda i: (0,)),
        ],
        out_specs=pl.BlockSpec((2048, D), lambda i: (i, 0)),
        out_shape=jax.ShapeDtypeStruct((N, D), jnp.float32),
    )(acc2, deg2, b)


def kernel(x, edge, edge_type, edge_norm, W, b):
    del edge_type, edge_norm
    deg2 = _deg_call(edge)
    hp = _mm_call(x, W, deg2)
    acc2 = _scat_call(hp, edge)
    return _fin_call(acc2, deg2, b)
